# baseline (speedup 1.0000x reference)
import functools

import jax
import jax.numpy as jnp
from jax import lax
from jax.experimental import pallas as pl
from jax.experimental.pallas import tpu as pltpu
from jax.experimental.pallas import tpu_sc as plsc

V = 1000000
D = 32
H = 128
O = 4
B = 1024
T = 20

N_IDX = B * T
_ROW = 128
_SUB = 16


def _sc_gather(idx3d, table_t):
  info = plsc.get_sparse_core_info()
  nw = info.num_cores * info.num_subcores
  b_per_w = N_IDX // nw
  n_chunk = b_per_w // _SUB
  mesh = plsc.VectorSubcoreMesh(core_axis_name="c", subcore_axis_name="s")

  @functools.partial(
      pl.kernel,
      mesh=mesh,
      out_type=jax.ShapeDtypeStruct((N_IDX, D), jnp.float32),
      scratch_types=[
          pltpu.VMEM((n_chunk, _SUB), jnp.int32),
          pltpu.VMEM((_SUB, D, 128), jnp.float32),
          pltpu.VMEM((2, _SUB, D), jnp.float32),
          pltpu.SemaphoreType.DMA,
          pltpu.SemaphoreType.DMA,
      ],
      compiler_params=pltpu.CompilerParams(needs_layout_passes=False),
  )
  def k(idx_hbm, table_hbm, out_hbm, idx_v, blk_v, rows_v, sem, sem2):
    wid = lax.axis_index("s") * info.num_cores + lax.axis_index("c")
    base = wid * b_per_w
    pltpu.sync_copy(idx_hbm.at[wid], idx_v)
    lanes = lax.iota(jnp.int32, 16)

    def chunk_body(i, _):
      slot = lax.rem(i, 2)
      chunk = idx_v[i, pl.ds(0, _SUB)]
      c16 = lax.rem(chunk, 128)
      a16 = chunk - c16
      copies = []
      for j in range(_SUB):
        a_j = jnp.max(jnp.where(lanes == j, a16, 0))
        copies.append(
            pltpu.async_copy(
                table_hbm.at[:, pl.ds(pl.multiple_of(a_j, 128), 128)],
                blk_v.at[j], sem))
      for cp in copies:
        cp.wait()

      @pl.when(i >= 2)
      def _():
        pltpu.make_async_copy(rows_v.at[slot],
                              out_hbm.at[pl.ds(base, _SUB)], sem2).wait()

      for j in range(_SUB):
        c_j = jnp.max(jnp.where(lanes == j, c16, 0))
        cs = jnp.full((16,), c_j, jnp.int32)
        js = jnp.full((16,), j, jnp.int32)
        for w in range(0, D, 16):
          val = plsc.load_gather(blk_v, [js, lanes + w, cs])
          rows_v[slot, j, pl.ds(w, 16)] = val
      pltpu.async_copy(rows_v.at[slot],
                       out_hbm.at[pl.ds(base + i * _SUB, _SUB)], sem2)
      return 0

    lax.fori_loop(0, n_chunk, chunk_body, 0)
    for _ in range(2):
      pltpu.make_async_copy(rows_v.at[0],
                            out_hbm.at[pl.ds(base, _SUB)], sem2).wait()

  return k(idx3d, table_t)


def _tc_lstm_head(E, Wf, Wr, bf, br, W1a, W1b, b1r, W2t, b2r):
  def body(e_ref, wf_ref, wr_ref, bf_ref, br_ref, w1a_ref, w1b_ref,
           b1_ref, w2_ref, b2_ref, out_ref,
           hf_ref, cf_ref, hb_ref, cb_ref, hf0_ref, hb0_ref):
    zeros = jnp.zeros((B, H), jnp.float32)
    hf_ref[...] = zeros
    cf_ref[...] = zeros
    hb_ref[...] = zeros
    cb_ref[...] = zeros

    def cell(x, h_ref, c_ref, w_ref, b_ref):
      xh = jnp.concatenate([x, h_ref[...]], axis=1).astype(jnp.bfloat16)
      gates = (jnp.dot(xh, w_ref[...],
                       preferred_element_type=jnp.float32) + b_ref[...])
      i = jax.nn.sigmoid(gates[:, 0 * H:1 * H])
      f = jax.nn.sigmoid(gates[:, 1 * H:2 * H])
      g = jnp.tanh(gates[:, 2 * H:3 * H])
      o = jax.nn.sigmoid(gates[:, 3 * H:4 * H])
      c_new = f * c_ref[...] + i * g
      h_new = o * jnp.tanh(c_new)
      c_ref[...] = c_new
      h_ref[...] = h_new
      return h_new

    def step(t, _):
      h_f = cell(e_ref[t], hf_ref, cf_ref, wf_ref, bf_ref)
      h_b = cell(e_ref[T - 1 - t], hb_ref, cb_ref, wr_ref, br_ref)

      @pl.when(t == 0)
      def _():
        hf0_ref[...] = h_f
        hb0_ref[...] = h_b

      return 0

    lax.fori_loop(0, T, step, 0)

    sf = hf0_ref[...] + hf_ref[...]
    sb = hb0_ref[...] + hb_ref[...]
    tmp = (jnp.dot(sf, w1a_ref[...], preferred_element_type=jnp.float32)
           + jnp.dot(sb, w1b_ref[...], preferred_element_type=jnp.float32)
           + b1_ref[...])
    out_ref[...] = (jnp.dot(tmp, w2_ref[...],
                            preferred_element_type=jnp.float32) + b2_ref[...])

  scratch = [pltpu.VMEM((B, H), jnp.float32)] * 6
  return pl.pallas_call(
      body,
      out_shape=jax.ShapeDtypeStruct((B, O), jnp.float32),
      scratch_shapes=scratch,
  )(E, Wf, Wr, bf, br, W1a, W1b, b1r, W2t, b2r)


def kernel(x, table, W_ih_f, W_hh_f, b_ih_f, b_hh_f,
           W_ih_r, W_hh_r, b_ih_r, b_hh_r, W1, b1, W2, b2):
  idx3d = x.astype(jnp.int32).T.reshape(32, -1, _SUB)
  E = _sc_gather(idx3d, table.T).reshape(T, B, D)

  Wf = jnp.concatenate([W_ih_f.T, W_hh_f.T], axis=0).astype(jnp.bfloat16)
  Wr = jnp.concatenate([W_ih_r.T, W_hh_r.T], axis=0).astype(jnp.bfloat16)
  bf = (b_ih_f + b_hh_f)[None, :]
  br = (b_ih_r + b_hh_r)[None, :]
  W1a = W1[:, :H].T
  W1b = W1[:, H:].T
  W2t = W2.T
  return _tc_lstm_head(E, Wf, Wr, bf, br, W1a, W1b, b1[None, :], W2t,
                       b2[None, :])


# speedup vs baseline: 1.0286x; 1.0286x over previous
import functools

import jax
import jax.numpy as jnp
from jax import lax
from jax.experimental import pallas as pl
from jax.experimental.pallas import tpu as pltpu
from jax.experimental.pallas import tpu_sc as plsc

V = 1000000
D = 32
H = 128
O = 4
B = 1024
T = 20

N_IDX = B * T
_ROW = 128
_SUB = 16


def _sc_gather(idx3d, table_t):
  info = plsc.get_sparse_core_info()
  nw = info.num_cores * info.num_subcores
  b_per_w = N_IDX // nw
  n_chunk = b_per_w // _SUB
  mesh = plsc.VectorSubcoreMesh(core_axis_name="c", subcore_axis_name="s")

  @functools.partial(
      pl.kernel,
      mesh=mesh,
      out_type=jax.ShapeDtypeStruct((N_IDX, D), jnp.float32),
      scratch_types=[
          pltpu.VMEM((n_chunk, _SUB), jnp.int32),
          pltpu.VMEM((_SUB, D, 128), jnp.float32),
          pltpu.VMEM((2, _SUB, D), jnp.float32),
          pltpu.SemaphoreType.DMA,
          pltpu.SemaphoreType.DMA,
      ],
      compiler_params=pltpu.CompilerParams(needs_layout_passes=False),
  )
  def k(idx_hbm, table_hbm, out_hbm, idx_v, blk_v, rows_v, sem, sem2):
    wid = lax.axis_index("s") * info.num_cores + lax.axis_index("c")
    base = wid * b_per_w
    pltpu.sync_copy(idx_hbm.at[wid], idx_v)
    lanes = lax.iota(jnp.int32, 16)

    def chunk_body(i, _):
      slot = lax.rem(i, 2)
      chunk = idx_v[i, pl.ds(0, _SUB)]
      c16 = lax.rem(chunk, 128)
      a16 = chunk - c16
      copies = []
      for j in range(_SUB):
        a_j = jnp.max(jnp.where(lanes == j, a16, 0))
        copies.append(
            pltpu.async_copy(
                table_hbm.at[:, pl.ds(pl.multiple_of(a_j, 128), 128)],
                blk_v.at[j], sem))
      for cp in copies:
        cp.wait()

      @pl.when(i >= 2)
      def _():
        pltpu.make_async_copy(rows_v.at[slot],
                              out_hbm.at[pl.ds(base, _SUB)], sem2).wait()

      for j in range(_SUB):
        c_j = jnp.max(jnp.where(lanes == j, c16, 0))
        cs = jnp.full((16,), c_j, jnp.int32)
        js = jnp.full((16,), j, jnp.int32)
        for w in range(0, D, 16):
          val = plsc.load_gather(blk_v, [js, lanes + w, cs])
          rows_v[slot, j, pl.ds(w, 16)] = val
      pltpu.async_copy(rows_v.at[slot],
                       out_hbm.at[pl.ds(base + i * _SUB, _SUB)], sem2)
      return 0

    lax.fori_loop(0, n_chunk, chunk_body, 0)
    for _ in range(2):
      pltpu.make_async_copy(rows_v.at[0],
                            out_hbm.at[pl.ds(base, _SUB)], sem2).wait()

  return k(idx3d, table_t)


def _tc_lstm_head(E, Wf, Wr, bf, br, W1a, W1b, b1r, W2t, b2r):
  def body(e_ref, wf_ref, wr_ref, bf_ref, br_ref, w1a_ref, w1b_ref,
           b1_ref, w2_ref, b2_ref, out_ref,
           hf_ref, cf_ref, hb_ref, cb_ref, hf0_ref, hb0_ref):
    zeros = jnp.zeros((B, H), jnp.float32)
    hf_ref[...] = zeros
    cf_ref[...] = zeros
    hb_ref[...] = zeros
    cb_ref[...] = zeros

    def sigmoid(z):
      return 0.5 + 0.5 * jnp.tanh(0.5 * z)

    def cell(x, h_ref, c_ref, w_ref, b_ref):
      xh = jnp.concatenate([x, h_ref[...]], axis=1)
      gates = (jnp.dot(xh, w_ref[...],
                       preferred_element_type=jnp.float32) + b_ref[...])
      i = sigmoid(gates[:, 0 * H:1 * H])
      f = sigmoid(gates[:, 1 * H:2 * H])
      g = jnp.tanh(gates[:, 2 * H:3 * H])
      o = sigmoid(gates[:, 3 * H:4 * H])
      c_new = f * c_ref[...] + i * g
      h_new = o * jnp.tanh(c_new)
      c_ref[...] = c_new
      h_ref[...] = h_new
      return h_new

    def step(t, _):
      h_f = cell(e_ref[t], hf_ref, cf_ref, wf_ref, bf_ref)
      h_b = cell(e_ref[T - 1 - t], hb_ref, cb_ref, wr_ref, br_ref)

      @pl.when(t == 0)
      def _():
        hf0_ref[...] = h_f
        hb0_ref[...] = h_b

      return 0

    lax.fori_loop(0, T, step, 0)

    sf = hf0_ref[...] + hf_ref[...]
    sb = hb0_ref[...] + hb_ref[...]
    tmp = (jnp.dot(sf, w1a_ref[...], preferred_element_type=jnp.float32)
           + jnp.dot(sb, w1b_ref[...], preferred_element_type=jnp.float32)
           + b1_ref[...])
    out_ref[...] = (jnp.dot(tmp, w2_ref[...],
                            preferred_element_type=jnp.float32) + b2_ref[...])

  scratch = [pltpu.VMEM((B, H), jnp.float32)] * 6
  return pl.pallas_call(
      body,
      out_shape=jax.ShapeDtypeStruct((B, O), jnp.float32),
      scratch_shapes=scratch,
  )(E, Wf, Wr, bf, br, W1a, W1b, b1r, W2t, b2r)


def kernel(x, table, W_ih_f, W_hh_f, b_ih_f, b_hh_f,
           W_ih_r, W_hh_r, b_ih_r, b_hh_r, W1, b1, W2, b2):
  idx3d = x.astype(jnp.int32).T.reshape(32, -1, _SUB)
  E = _sc_gather(idx3d, table.T).reshape(T, B, D)

  Wf = jnp.concatenate([W_ih_f.T, W_hh_f.T], axis=0)
  Wr = jnp.concatenate([W_ih_r.T, W_hh_r.T], axis=0)
  bf = (b_ih_f + b_hh_f)[None, :]
  br = (b_ih_r + b_hh_r)[None, :]
  W1a = W1[:, :H].T
  W1b = W1[:, H:].T
  W2t = W2.T
  return _tc_lstm_head(E, Wf, Wr, bf, br, W1a, W1b, b1[None, :], W2t,
                       b2[None, :])


# speedup vs baseline: 1.0304x; 1.0017x over previous
import functools

import jax
import jax.numpy as jnp
from jax import lax
from jax.experimental import pallas as pl
from jax.experimental.pallas import tpu as pltpu
from jax.experimental.pallas import tpu_sc as plsc

V = 1000000
D = 32
H = 128
O = 4
B = 1024
T = 20

N_IDX = B * T
_SUB = 16


def _sc_gather(idx3d, table_t):
  info = plsc.get_sparse_core_info()
  nw = info.num_cores * info.num_subcores
  b_per_w = N_IDX // nw
  n_chunk = b_per_w // _SUB
  mesh = plsc.VectorSubcoreMesh(core_axis_name="c", subcore_axis_name="s")

  @functools.partial(
      pl.kernel,
      mesh=mesh,
      out_type=jax.ShapeDtypeStruct((N_IDX, D), jnp.float32),
      scratch_types=[
          pltpu.VMEM((n_chunk, _SUB), jnp.int32),
          pltpu.VMEM((_SUB, D, 128), jnp.float32),
          pltpu.VMEM((2, _SUB, D), jnp.float32),
          pltpu.SemaphoreType.DMA,
          pltpu.SemaphoreType.DMA,
      ],
      compiler_params=pltpu.CompilerParams(needs_layout_passes=False),
  )
  def k(idx_hbm, table_hbm, out_hbm, idx_v, blk_v, rows_v, sem, sem2):
    wid = lax.axis_index("s") * info.num_cores + lax.axis_index("c")
    base = wid * b_per_w
    pltpu.sync_copy(idx_hbm.at[wid], idx_v)
    lanes = lax.iota(jnp.int32, 16)

    def chunk_body(i, _):
      slot = lax.rem(i, 2)
      chunk = idx_v[i, pl.ds(0, _SUB)]
      c16 = lax.rem(chunk, 128)
      a16 = chunk - c16
      copies = []
      for j in range(_SUB):
        a_j = jnp.max(jnp.where(lanes == j, a16, 0))
        copies.append(
            pltpu.async_copy(
                table_hbm.at[:, pl.ds(pl.multiple_of(a_j, 128), 128)],
                blk_v.at[j], sem))
      for cp in copies:
        cp.wait()

      @pl.when(i >= 2)
      def _():
        pltpu.make_async_copy(rows_v.at[slot],
                              out_hbm.at[pl.ds(base, _SUB)], sem2).wait()

      for j in range(_SUB):
        c_j = jnp.max(jnp.where(lanes == j, c16, 0))
        cs = jnp.full((16,), c_j, jnp.int32)
        js = jnp.full((16,), j, jnp.int32)
        for w in range(0, D, 16):
          val = plsc.load_gather(blk_v, [js, lanes + w, cs])
          rows_v[slot, j, pl.ds(w, 16)] = val
      pltpu.async_copy(rows_v.at[slot],
                       out_hbm.at[pl.ds(base + i * _SUB, _SUB)], sem2)
      return 0

    lax.fori_loop(0, n_chunk, chunk_body, 0)
    for _ in range(2):
      pltpu.make_async_copy(rows_v.at[0],
                            out_hbm.at[pl.ds(base, _SUB)], sem2).wait()

  return k(idx3d, table_t)


def _tc_lstm_head(E, Wf, Wr, bf, br, W1a, W1b, b1r, W2t, b2r):
  def body(e_ref, wf_ref, wr_ref, bf_ref, br_ref, w1a_ref, w1b_ref,
           b1_ref, w2_ref, b2_ref, out_ref,
           hf_ref, cf_ref, hb_ref, cb_ref, hf0_ref, hb0_ref):
    zeros = jnp.zeros((B, H), jnp.float32)
    hf_ref[...] = zeros
    cf_ref[...] = zeros
    hb_ref[...] = zeros
    cb_ref[...] = zeros

    def sigmoid(z):
      return 0.5 + 0.5 * jnp.tanh(0.5 * z)

    def cell(x, h_ref, c_ref, w_ref, b_ref):
      xh = jnp.concatenate([x, h_ref[...]], axis=1)
      gates = (jnp.dot(xh, w_ref[...],
                       preferred_element_type=jnp.float32) + b_ref[...])
      i = sigmoid(gates[:, 0 * H:1 * H])
      f = sigmoid(gates[:, 1 * H:2 * H])
      g = jnp.tanh(gates[:, 2 * H:3 * H])
      o = sigmoid(gates[:, 3 * H:4 * H])
      c_new = f * c_ref[...] + i * g
      h_new = o * jnp.tanh(c_new)
      c_ref[...] = c_new
      h_ref[...] = h_new
      return h_new

    def step(t, _):
      h_f = cell(e_ref[t], hf_ref, cf_ref, wf_ref, bf_ref)
      h_b = cell(e_ref[T - 1 - t], hb_ref, cb_ref, wr_ref, br_ref)

      @pl.when(t == 0)
      def _():
        hf0_ref[...] = h_f
        hb0_ref[...] = h_b

      return 0

    lax.fori_loop(0, T, step, 0)

    sf = hf0_ref[...] + hf_ref[...]
    sb = hb0_ref[...] + hb_ref[...]
    tmp = (jnp.dot(sf, w1a_ref[...], preferred_element_type=jnp.float32)
           + jnp.dot(sb, w1b_ref[...], preferred_element_type=jnp.float32)
           + b1_ref[...])
    out_ref[...] = (jnp.dot(tmp, w2_ref[...],
                            preferred_element_type=jnp.float32) + b2_ref[...])

  scratch = [pltpu.VMEM((B, H), jnp.float32)] * 6
  return pl.pallas_call(
      body,
      out_shape=jax.ShapeDtypeStruct((B, O), jnp.float32),
      scratch_shapes=scratch,
  )(E, Wf, Wr, bf, br, W1a, W1b, b1r, W2t, b2r)


def kernel(x, table, W_ih_f, W_hh_f, b_ih_f, b_hh_f,
           W_ih_r, W_hh_r, b_ih_r, b_hh_r, W1, b1, W2, b2):
  idx3d = x.astype(jnp.int32).T.reshape(32, -1, _SUB)
  E = _sc_gather(idx3d, table.T).reshape(T, B, D)

  Wf = jnp.concatenate([W_ih_f.T, W_hh_f.T], axis=0)
  Wr = jnp.concatenate([W_ih_r.T, W_hh_r.T], axis=0)
  bf = (b_ih_f + b_hh_f)[None, :]
  br = (b_ih_r + b_hh_r)[None, :]
  W1a = W1[:, :H].T
  W1b = W1[:, H:].T
  W2t = W2.T
  return _tc_lstm_head(E, Wf, Wr, bf, br, W1a, W1b, b1[None, :], W2t,
                       b2[None, :])
